# baseline (speedup 1.0000x reference)
import functools

import jax
import jax.numpy as jnp
from jax import lax
from jax.experimental import pallas as pl
from jax.experimental.pallas import tpu as pltpu
from jax.experimental.pallas import tpu_sc as plsc

N = 10000
E = 320000
D = 128

NC = 2
NS = 16
NW = NC * NS

EPW = E // NW
C = 80
NCHUNK = EPW // C
NB = 4
NI = 8
RPT = 624
TAIL = N - NS * RPT


def _sc_aggregate(edge_index, w, x):
    mesh = plsc.VectorSubcoreMesh(
        core_axis_name="c", subcore_axis_name="s", num_cores=NC, num_subcores=NS
    )

    @functools.partial(
        pl.kernel,
        out_type=jax.ShapeDtypeStruct((NC, N, D), jnp.float32),
        mesh=mesh,
        scratch_types=[
            pltpu.VMEM((NI, C), jnp.int32),
            pltpu.VMEM((NI, C), jnp.int32),
            pltpu.VMEM((NI, C), jnp.float32),
            pltpu.VMEM((NB, C, D), jnp.float32),
            pltpu.VMEM_SHARED((N, D), jnp.float32),
            pltpu.SemaphoreType.DMA((NI,)),
            pltpu.SemaphoreType.DMA((NB,)),
            pltpu.SemaphoreType.DMA((NB,)),
        ],
    )
    def agg_kernel(ei_hbm, w_hbm, x_hbm, out_hbm,
                   src_c, dst_c, w_c, rows_v, acc, isems, gsems, ssems):
        cid = lax.axis_index("c")
        sid = lax.axis_index("s")
        wid = cid * NS + sid

        zero = jnp.zeros((16,), jnp.float32)

        def zero_row(i, carry):
            for s in range(D // 16):
                rows_v[0, i, pl.ds(16 * s, 16)] = zero
            return carry

        lax.fori_loop(0, C, zero_row, 0)

        base = pl.multiple_of(sid * RPT, 8)
        done = 0
        while done < RPT:
            step = min(C, RPT - done)
            pltpu.sync_copy(rows_v.at[0, pl.ds(0, step)],
                            acc.at[pl.ds(base + done, step)])
            done += step

        @pl.when(sid < 2)
        def _zero_tail():
            tail_base = pl.multiple_of(NS * RPT + sid * 8, 8)
            pltpu.sync_copy(rows_v.at[0, pl.ds(0, 8)],
                            acc.at[pl.ds(tail_base, 8)])

        plsc.subcore_barrier()

        def idx_start(k):
            bi = lax.bitwise_and(k, NI - 1)
            off = pl.multiple_of(wid * EPW + k * C, 8)
            pltpu.async_copy(ei_hbm.at[pl.ds(E + off, C)], src_c.at[bi],
                             isems.at[bi])
            pltpu.async_copy(ei_hbm.at[pl.ds(off, C)], dst_c.at[bi],
                             isems.at[bi])
            pltpu.async_copy(w_hbm.at[pl.ds(off, C)], w_c.at[bi],
                             isems.at[bi])

        def idx_wait(k):
            bi = lax.bitwise_and(k, NI - 1)
            off = pl.multiple_of(wid * EPW + k * C, 8)
            pltpu.make_async_copy(ei_hbm.at[pl.ds(E + off, C)], src_c.at[bi],
                                  isems.at[bi]).wait()
            pltpu.make_async_copy(ei_hbm.at[pl.ds(off, C)], dst_c.at[bi],
                                  isems.at[bi]).wait()
            pltpu.make_async_copy(w_hbm.at[pl.ds(off, C)], w_c.at[bi],
                                  isems.at[bi]).wait()

        def gather_start(k):
            bi = lax.bitwise_and(k, NI - 1)
            b = lax.bitwise_and(k, NB - 1)
            pltpu.async_copy(x_hbm.at[src_c.at[bi]], rows_v.at[b],
                             gsems.at[b])

        def gather_wait(k):
            bi = lax.bitwise_and(k, NI - 1)
            b = lax.bitwise_and(k, NB - 1)
            pltpu.make_async_copy(x_hbm.at[src_c.at[bi]], rows_v.at[b],
                                  gsems.at[b]).wait()

        def compute(k):
            bi = lax.bitwise_and(k, NI - 1)
            b = lax.bitwise_and(k, NB - 1)

            def group_body(g, c2):
                w16 = w_c[bi, pl.ds(g * 16, 16)]
                for j in range(16):
                    wv = jnp.full((16,), w16[j], jnp.float32)
                    e = g * 16 + j
                    for s in range(D // 16):
                        rows_v[b, e, pl.ds(16 * s, 16)] = (
                            rows_v[b, e, pl.ds(16 * s, 16)] * wv
                        )
                return c2

            lax.fori_loop(0, C // 16, group_body, 0)

        def scatter_start(k):
            bi = lax.bitwise_and(k, NI - 1)
            b = lax.bitwise_and(k, NB - 1)
            pltpu.async_copy(rows_v.at[b], acc.at[dst_c.at[bi]],
                             ssems.at[b], add=True)

        def scatter_wait(k):
            bi = lax.bitwise_and(k, NI - 1)
            b = lax.bitwise_and(k, NB - 1)
            pltpu.make_async_copy(rows_v.at[b], acc.at[dst_c.at[bi]],
                                  ssems.at[b]).wait()

        for k in (0, 1, 2, 3):
            idx_start(jnp.int32(k))
        for k in (0, 1):
            idx_wait(jnp.int32(k))
            gather_start(jnp.int32(k))

        def loop_body(k, carry):
            @pl.when(k + 4 < NCHUNK)
            def _prefetch_idx():
                idx_start(k + 4)

            @pl.when(k + 2 < NCHUNK)
            def _prefetch_gather():
                @pl.when(k >= 2)
                def _drain():
                    scatter_wait(k - 2)

                idx_wait(k + 2)
                gather_start(k + 2)

            gather_wait(k)
            compute(k)
            scatter_start(k)
            return carry

        lax.fori_loop(0, NCHUNK, loop_body, 0)

        for m in range(NCHUNK - 4, NCHUNK):
            scatter_wait(jnp.int32(m))

        plsc.subcore_barrier()

        pltpu.sync_copy(acc.at[pl.ds(base, RPT)],
                        out_hbm.at[cid, pl.ds(base, RPT)])

        @pl.when(sid < 2)
        def _write_tail():
            tail_base = pl.multiple_of(NS * RPT + sid * 8, 8)
            pltpu.sync_copy(acc.at[pl.ds(tail_base, 8)],
                            out_hbm.at[cid, pl.ds(tail_base, 8)])

    return agg_kernel(edge_index, w, x)


def _tc_linear(partials, W, b2d):
    BN = 1000

    def body(p_ref, w_ref, b_ref, o_ref):
        a = p_ref[0] + p_ref[1]
        o_ref[...] = (
            lax.dot_general(a, w_ref[...], (((1,), (1,)), ((), ())),
                            preferred_element_type=jnp.float32)
            + b_ref[...]
        )

    return pl.pallas_call(
        body,
        grid=(N // BN,),
        in_specs=[
            pl.BlockSpec((NC, BN, D), lambda i: (0, i, 0)),
            pl.BlockSpec((D, D), lambda i: (0, 0)),
            pl.BlockSpec((1, D), lambda i: (0, 0)),
        ],
        out_specs=pl.BlockSpec((BN, D), lambda i: (i, 0)),
        out_shape=jax.ShapeDtypeStruct((N, D), jnp.float32),
    )(partials, W, b2d)


def kernel(x, edge_index, edge_weight, W, b):
    partials = _sc_aggregate(edge_index.astype(jnp.int32).reshape(2 * E),
                             edge_weight, x)
    return _tc_linear(partials, W, b.reshape(1, D))


# speedup vs baseline: 2.8385x; 2.8385x over previous
import functools

import jax
import jax.numpy as jnp
from jax import lax
from jax.experimental import pallas as pl
from jax.experimental.pallas import tpu as pltpu
from jax.experimental.pallas import tpu_sc as plsc

N = 10000
E = 320000
D = 128

NC = 2
NS = 16
NW = NC * NS

EPW = E // NW
C = 80
NCHUNK = EPW // C
NB = 4
NI = 8
RPT = 624
TAIL = N - NS * RPT


def _sc_aggregate(ei_flat, w, x):
    mesh = plsc.VectorSubcoreMesh(
        core_axis_name="c", subcore_axis_name="s", num_cores=NC, num_subcores=NS
    )

    @functools.partial(
        pl.kernel,
        out_type=jax.ShapeDtypeStruct((NC, N, D), jnp.float32),
        mesh=mesh,
        scratch_types=[
            pltpu.VMEM((NI, C), jnp.int32),
            pltpu.VMEM((NI, C), jnp.int32),
            pltpu.VMEM((NI, C), jnp.float32),
            pltpu.VMEM((NB, C, D), jnp.float32),
            pltpu.VMEM_SHARED((N, D), jnp.float32),
            [pltpu.SemaphoreType.DMA] * NI,
            [pltpu.SemaphoreType.DMA] * NB,
            [pltpu.SemaphoreType.DMA] * NB,
        ],
    )
    def agg_kernel(ei_hbm, w_hbm, x_hbm, out_hbm,
                   src_c, dst_c, w_c, rows_v, acc, isems, gsems, ssems):
        cid = lax.axis_index("c")
        sid = lax.axis_index("s")
        wid = cid * NS + sid

        def idx_start(k, bi):
            off = pl.multiple_of(wid * EPW + k * C, 8)
            pltpu.async_copy(ei_hbm.at[pl.ds(E + off, C)], src_c.at[bi],
                             isems[bi])
            pltpu.async_copy(ei_hbm.at[pl.ds(off, C)], dst_c.at[bi],
                             isems[bi])
            pltpu.async_copy(w_hbm.at[pl.ds(off, C)], w_c.at[bi], isems[bi])

        def idx_wait(k, bi):
            off = pl.multiple_of(wid * EPW + k * C, 8)
            pltpu.make_async_copy(ei_hbm.at[pl.ds(E + off, C)], src_c.at[bi],
                                  isems[bi]).wait()
            pltpu.make_async_copy(ei_hbm.at[pl.ds(off, C)], dst_c.at[bi],
                                  isems[bi]).wait()
            pltpu.make_async_copy(w_hbm.at[pl.ds(off, C)], w_c.at[bi],
                                  isems[bi]).wait()

        def gather_start(bi, b):
            pltpu.async_copy(x_hbm.at[src_c.at[bi]], rows_v.at[b], gsems[b])

        def gather_wait(bi, b):
            pltpu.make_async_copy(x_hbm.at[src_c.at[bi]], rows_v.at[b],
                                  gsems[b]).wait()

        for k in (0, 1, 2, 3):
            idx_start(k, k)
        idx_wait(1, 1)
        gather_start(1, 1)
        idx_wait(2, 2)
        gather_start(2, 2)

        zero = jnp.zeros((16,), jnp.float32)

        def zero_row(i, carry):
            for s in range(D // 16):
                rows_v[0, i, pl.ds(16 * s, 16)] = zero
            return carry

        lax.fori_loop(0, C, zero_row, 0)

        base = pl.multiple_of(sid * RPT, 8)
        done = 0
        while done < RPT:
            step = min(C, RPT - done)
            pltpu.sync_copy(rows_v.at[0, pl.ds(0, step)],
                            acc.at[pl.ds(base + done, step)])
            done += step

        @pl.when(sid < 2)
        def _zero_tail():
            tail_base = pl.multiple_of(NS * RPT + sid * 8, 8)
            pltpu.sync_copy(rows_v.at[0, pl.ds(0, 8)],
                            acc.at[pl.ds(tail_base, 8)])

        plsc.subcore_barrier()

        def compute(bi, b):
            def group_body(g, c2):
                w16 = w_c[bi, pl.ds(g * 16, 16)]
                for j in range(16):
                    wv = jnp.full((16,), w16[j], jnp.float32)
                    e = g * 16 + j
                    for s in range(D // 16):
                        rows_v[b, e, pl.ds(16 * s, 16)] = (
                            rows_v[b, e, pl.ds(16 * s, 16)] * wv
                        )
                return c2

            lax.fori_loop(0, C // 16, group_body, 0)

        def scatter_start(bi, b):
            pltpu.async_copy(rows_v.at[b], acc.at[dst_c.at[bi]], ssems[b],
                             add=True)

        def scatter_wait(bi, b):
            pltpu.make_async_copy(rows_v.at[b], acc.at[dst_c.at[bi]],
                                  ssems[b]).wait()

        idx_wait(0, 0)
        gather_start(0, 0)
        idx_start(4, 4)
        idx_wait(3, 3)
        gather_start(3, 3)
        gather_wait(0, 0)
        compute(0, 0)
        scatter_start(0, 0)
        idx_start(5, 5)
        gather_wait(1, 1)
        compute(1, 1)
        scatter_start(1, 1)

        def oct_body(i, carry):
            for j in range(8):
                k = 2 + 8 * i + j
                bi, b = (2 + j) % NI, (2 + j) % NB
                bi4 = (6 + j) % NI
                bi2, b2 = (4 + j) % NI, j % NB

                @pl.when(k <= NCHUNK - 5)
                def _prefetch_idx():
                    idx_start(k + 4, bi4)

                scatter_wait(j % NI, j % NB)
                idx_wait(k + 2, bi2)
                gather_start(bi2, b2)
                gather_wait(bi, b)
                compute(bi, b)
                scatter_start(bi, b)
            return carry

        lax.fori_loop(0, (NCHUNK - 5) // 8, oct_body, 0)

        scatter_wait(120 % NI, 120 % NB)
        idx_wait(124, 124 % NI)
        gather_start(124 % NI, 124 % NB)
        gather_wait(122 % NI, 122 % NB)
        compute(122 % NI, 122 % NB)
        scatter_start(122 % NI, 122 % NB)
        gather_wait(123 % NI, 123 % NB)
        compute(123 % NI, 123 % NB)
        scatter_start(123 % NI, 123 % NB)
        gather_wait(124 % NI, 124 % NB)
        compute(124 % NI, 124 % NB)
        scatter_start(124 % NI, 124 % NB)
        for m in (121, 122, 123, 124):
            scatter_wait(m % NI, m % NB)

        plsc.subcore_barrier()

        pltpu.sync_copy(acc.at[pl.ds(base, RPT)],
                        out_hbm.at[cid, pl.ds(base, RPT)])

        @pl.when(sid < 2)
        def _write_tail():
            tail_base = pl.multiple_of(NS * RPT + sid * 8, 8)
            pltpu.sync_copy(acc.at[pl.ds(tail_base, 8)],
                            out_hbm.at[cid, pl.ds(tail_base, 8)])

    return agg_kernel(ei_flat, w, x)


def _tc_linear(partials, W, b2d):
    BN = 1000

    def body(p_ref, w_ref, b_ref, o_ref):
        a = p_ref[0] + p_ref[1]
        o_ref[...] = (
            lax.dot_general(a, w_ref[...], (((1,), (1,)), ((), ())),
                            preferred_element_type=jnp.float32)
            + b_ref[...]
        )

    return pl.pallas_call(
        body,
        grid=(N // BN,),
        in_specs=[
            pl.BlockSpec((NC, BN, D), lambda i: (0, i, 0)),
            pl.BlockSpec((D, D), lambda i: (0, 0)),
            pl.BlockSpec((1, D), lambda i: (0, 0)),
        ],
        out_specs=pl.BlockSpec((BN, D), lambda i: (i, 0)),
        out_shape=jax.ShapeDtypeStruct((N, D), jnp.float32),
    )(partials, W, b2d)


def kernel(x, edge_index, edge_weight, W, b):
    ei_flat = edge_index.astype(jnp.int32).reshape(2 * E)
    partials = _sc_aggregate(ei_flat, edge_weight, x)
    return _tc_linear(partials, W, b.reshape(1, D))
